# Initial kernel scaffold; baseline (speedup 1.0000x reference)
import jax
import jax.numpy as jnp
from jax.experimental import pallas as pl


def kernel(user_emb, item_emb, edge_src, edge_dst, edge_val):
    raise NotImplementedError("write your pallas kernel here")


# speedup vs baseline: 2.9492x; 2.9492x over previous
import jax
import jax.numpy as jnp
from jax import lax
from jax.experimental import pallas as pl
from jax.experimental.pallas import tpu as pltpu
from jax.experimental.pallas import tpu_sc as plsc

_N_NODES = 100000
_HALF = 50000
_EMB = 32
_E = 1600000
_NC = 2
_NS = 16
_K = 128
_EPT = _E // _NS
_NFULL = _EPT // _K
_TAIL = _EPT - _NFULL * _K
_PAD = 51200
_TRASH = 50048
_ZSPAN = _PAD // _NS
_CSPAN = 3128
_CLAST = _HALF - 15 * _CSPAN


def _scale_rows(rows_ref, val_ref, zidx_ref, k):
    for e in range(k):
        if e == 0:
            idx = zidx_ref[...]
        else:
            idx = jnp.full((16,), e, jnp.int32)
        vs = plsc.load_gather(val_ref, [idx])
        rows_ref[e, pl.ds(0, 16)] = rows_ref[e, pl.ds(0, 16)] * vs
        rows_ref[e, pl.ds(16, 16)] = rows_ref[e, pl.ds(16, 16)] * vs


def _localize_src(srcloc_ref, base, k):
    for g in range(k // 16):
        s = srcloc_ref[pl.ds(g * 16, 16)] - base
        ok = (s >= 0) & (s < _HALF)
        srcloc_ref[pl.ds(g * 16, 16)] = jnp.where(ok, s, _TRASH)


def _prop_body(emb, src, dst, val, out,
               dst_v, val_v, srcloc_v, rows_v,
               dst_t, val_t, srcloc_t, rows_t, zidx_v, acc):
    cid = lax.axis_index("c")
    sid = lax.axis_index("s")
    base = cid * _HALF
    zidx_v[...] = jnp.zeros((16,), jnp.int32)

    z = jnp.zeros((16,), jnp.float32)

    def zrow(i, carry):
        rows_v[i, pl.ds(0, 16)] = z
        rows_v[i, pl.ds(16, 16)] = z
        return carry

    lax.fori_loop(0, _K, zrow, 0)

    def zacc(j, carry):
        pltpu.sync_copy(rows_v, acc.at[pl.ds(sid * _ZSPAN + j * _K, _K)])
        return carry

    lax.fori_loop(0, _ZSPAN // _K, zacc, 0)
    plsc.subcore_barrier()

    ebase = sid * _EPT

    def chunk(i, carry):
        off = ebase + i * _K
        pltpu.sync_copy(dst.at[pl.ds(off, _K)], dst_v)
        pltpu.sync_copy(val.at[pl.ds(off, _K)], val_v)
        pltpu.sync_copy(src.at[pl.ds(off, _K)], srcloc_v)
        pltpu.sync_copy(emb.at[dst_v], rows_v)
        _localize_src(srcloc_v, base, _K)
        _scale_rows(rows_v, val_v, zidx_v, _K)
        pltpu.sync_copy(rows_v, acc.at[srcloc_v], add=True)
        return carry

    lax.fori_loop(0, _NFULL, chunk, 0)

    toff = ebase + _NFULL * _K
    pltpu.sync_copy(dst.at[pl.ds(toff, _TAIL)], dst_t)
    pltpu.sync_copy(val.at[pl.ds(toff, _TAIL)], val_t)
    pltpu.sync_copy(src.at[pl.ds(toff, _TAIL)], srcloc_t)
    pltpu.sync_copy(emb.at[dst_t], rows_t)
    _localize_src(srcloc_t, base, _TAIL)
    _scale_rows(rows_t, val_t, zidx_v, _TAIL)
    pltpu.sync_copy(rows_t, acc.at[srcloc_t], add=True)

    plsc.subcore_barrier()

    @pl.when(sid < _NS - 1)
    def _copy_main():
        pltpu.sync_copy(acc.at[pl.ds(sid * _CSPAN, _CSPAN)],
                        out.at[pl.ds(base + sid * _CSPAN, _CSPAN)])

    @pl.when(sid == _NS - 1)
    def _copy_last():
        pltpu.sync_copy(acc.at[pl.ds(15 * _CSPAN, _CLAST)],
                        out.at[pl.ds(base + 15 * _CSPAN, _CLAST)])


def _propagate(emb, src, dst, val):
    mesh = plsc.VectorSubcoreMesh(core_axis_name="c", subcore_axis_name="s",
                                  num_cores=_NC, num_subcores=_NS)
    f = pl.kernel(
        _prop_body,
        out_type=jax.ShapeDtypeStruct((_N_NODES, _EMB), jnp.float32),
        mesh=mesh,
        scratch_types=[
            pltpu.VMEM((_K,), jnp.int32),
            pltpu.VMEM((_K,), jnp.float32),
            pltpu.VMEM((_K,), jnp.int32),
            pltpu.VMEM((_K, _EMB), jnp.float32),
            pltpu.VMEM((_TAIL,), jnp.int32),
            pltpu.VMEM((_TAIL,), jnp.float32),
            pltpu.VMEM((_TAIL,), jnp.int32),
            pltpu.VMEM((_TAIL, _EMB), jnp.float32),
            pltpu.VMEM((16,), jnp.int32),
            pltpu.VMEM_SHARED((_PAD, _EMB), jnp.float32),
        ],
        compiler_params=pltpu.CompilerParams(use_tc_tiling_on_sc=False,
                                             needs_layout_passes=False),
    )
    return f(emb, src, dst, val)


def _mean_body(a_ref, b_ref, c_ref, d_ref, o_ref):
    o_ref[...] = (a_ref[...] + b_ref[...] + c_ref[...] + d_ref[...]) * 0.25


def _mean4(a, b, c, d):
    blk = (2000, _EMB)
    spec = pl.BlockSpec(blk, lambda i: (i, 0))
    return pl.pallas_call(
        _mean_body,
        grid=(_N_NODES // blk[0],),
        in_specs=[spec] * 4,
        out_specs=spec,
        out_shape=jax.ShapeDtypeStruct((_N_NODES, _EMB), jnp.float32),
    )(a, b, c, d)


def kernel(user_emb, item_emb, edge_src, edge_dst, edge_val):
    e0 = jnp.concatenate([user_emb, item_emb], axis=0)
    e1 = _propagate(e0, edge_src, edge_dst, edge_val)
    e2 = _propagate(e1, edge_src, edge_dst, edge_val)
    e3 = _propagate(e2, edge_src, edge_dst, edge_val)
    m = _mean4(e0, e1, e2, e3)
    return m[:_HALF], m[_HALF:]


# speedup vs baseline: 4.6707x; 1.5837x over previous
import jax
import jax.numpy as jnp
from jax import lax
from jax.experimental import pallas as pl
from jax.experimental.pallas import tpu as pltpu
from jax.experimental.pallas import tpu_sc as plsc

_N_NODES = 100000
_HALF = 50000
_EMB = 32
_E = 1600000
_NC = 2
_NS = 16
_K = 128
_NBUF = 4
_BLK = _K * _NBUF
_EPT = _E // _NS
_NBLK = _EPT // _BLK
_NREM = (_EPT - _NBLK * _BLK) // _K
_TAIL = _EPT - _NBLK * _BLK - _NREM * _K
_PAD = 51200
_TRASH = 50048
_ZSPAN = _PAD // _NS
_CSPAN = 3128
_CLAST = _HALF - 15 * _CSPAN


def _scale_rows(rows_ref, val_ref, zidx_ref, k, vbase=0):
    for e in range(k):
        if vbase + e == 0:
            idx = zidx_ref[...]
        else:
            idx = jnp.full((16,), vbase + e, jnp.int32)
        vs = plsc.load_gather(val_ref, [idx])
        rows_ref[e, pl.ds(0, 16)] = rows_ref[e, pl.ds(0, 16)] * vs
        rows_ref[e, pl.ds(16, 16)] = rows_ref[e, pl.ds(16, 16)] * vs


def _localize_src(srcloc_ref, base, k, src_ref=None, sbase=0):
    rd = srcloc_ref if src_ref is None else src_ref
    for g in range(k // 16):
        s = rd[pl.ds(sbase + g * 16, 16)] - base
        ok = (s >= 0) & (s < _HALF)
        srcloc_ref[pl.ds(g * 16, 16)] = jnp.where(ok, s, _TRASH)


def _prop_body(emb, src, dst, val, out,
               dstb, valb, srcb, rows0, rows1, rows2, rows3,
               sl0, sl1, sl2, sl3,
               dst_t, val_t, srcloc_t, rows_t, zidx_v,
               g0, g1, g2, g3, s0, s1, s2, s3, acc):
    cid = lax.axis_index("c")
    sid = lax.axis_index("s")
    base = cid * _HALF
    zidx_v[...] = jnp.zeros((16,), jnp.int32)
    rows = [rows0, rows1, rows2, rows3]
    sls = [sl0, sl1, sl2, sl3]
    gsems = [g0, g1, g2, g3]
    ssems = [s0, s1, s2, s3]

    z = jnp.zeros((16,), jnp.float32)

    def zrow(i, carry):
        rows0[i, pl.ds(0, 16)] = z
        rows0[i, pl.ds(16, 16)] = z
        return carry

    lax.fori_loop(0, _K, zrow, 0)

    def zacc(j, carry):
        pltpu.sync_copy(rows0, acc.at[pl.ds(sid * _ZSPAN + j * _K, _K)])
        return carry

    lax.fori_loop(0, _ZSPAN // _K, zacc, 0)
    plsc.subcore_barrier()

    ebase = sid * _EPT

    def block(i, carry):
        off = ebase + i * _BLK
        pltpu.sync_copy(dst.at[pl.ds(off, _BLK)], dstb)
        pltpu.sync_copy(val.at[pl.ds(off, _BLK)], valb)
        pltpu.sync_copy(src.at[pl.ds(off, _BLK)], srcb)
        gd = [pltpu.async_copy(emb.at[dstb.at[pl.ds(j * _K, _K)]],
                               rows[j], gsems[j])
              for j in range(_NBUF)]
        sd = []
        for j in range(_NBUF):
            gd[j].wait()
            _localize_src(sls[j], base, _K, src_ref=srcb, sbase=j * _K)
            _scale_rows(rows[j], valb, zidx_v, _K, vbase=j * _K)
            sd.append(pltpu.async_copy(rows[j], acc.at[sls[j]],
                                       ssems[j], add=True))
        for d in sd:
            d.wait()
        return carry

    lax.fori_loop(0, _NBLK, block, 0)

    for r in range(_NREM):
        roff = ebase + _NBLK * _BLK + r * _K
        pltpu.sync_copy(dst.at[pl.ds(roff, _K)], dstb.at[pl.ds(0, _K)])
        pltpu.sync_copy(val.at[pl.ds(roff, _K)], valb.at[pl.ds(0, _K)])
        pltpu.sync_copy(src.at[pl.ds(roff, _K)], srcb.at[pl.ds(0, _K)])
        pltpu.sync_copy(emb.at[dstb.at[pl.ds(0, _K)]], rows0)
        _localize_src(sl0, base, _K, src_ref=srcb, sbase=0)
        _scale_rows(rows0, valb, zidx_v, _K)
        pltpu.sync_copy(rows0, acc.at[sl0], add=True)

    toff = ebase + _NBLK * _BLK + _NREM * _K
    pltpu.sync_copy(dst.at[pl.ds(toff, _TAIL)], dst_t)
    pltpu.sync_copy(val.at[pl.ds(toff, _TAIL)], val_t)
    pltpu.sync_copy(src.at[pl.ds(toff, _TAIL)], srcloc_t)
    pltpu.sync_copy(emb.at[dst_t], rows_t)
    _localize_src(srcloc_t, base, _TAIL)
    _scale_rows(rows_t, val_t, zidx_v, _TAIL)
    pltpu.sync_copy(rows_t, acc.at[srcloc_t], add=True)

    plsc.subcore_barrier()

    @pl.when(sid < _NS - 1)
    def _copy_main():
        pltpu.sync_copy(acc.at[pl.ds(sid * _CSPAN, _CSPAN)],
                        out.at[pl.ds(base + sid * _CSPAN, _CSPAN)])

    @pl.when(sid == _NS - 1)
    def _copy_last():
        pltpu.sync_copy(acc.at[pl.ds(15 * _CSPAN, _CLAST)],
                        out.at[pl.ds(base + 15 * _CSPAN, _CLAST)])


def _propagate(emb, src, dst, val):
    mesh = plsc.VectorSubcoreMesh(core_axis_name="c", subcore_axis_name="s",
                                  num_cores=_NC, num_subcores=_NS)
    f = pl.kernel(
        _prop_body,
        out_type=jax.ShapeDtypeStruct((_N_NODES, _EMB), jnp.float32),
        mesh=mesh,
        scratch_types=[
            pltpu.VMEM((_BLK,), jnp.int32),
            pltpu.VMEM((_BLK,), jnp.float32),
            pltpu.VMEM((_BLK,), jnp.int32),
        ] + [pltpu.VMEM((_K, _EMB), jnp.float32)] * _NBUF
          + [pltpu.VMEM((_K,), jnp.int32)] * _NBUF
          + [
            pltpu.VMEM((_TAIL,), jnp.int32),
            pltpu.VMEM((_TAIL,), jnp.float32),
            pltpu.VMEM((_TAIL,), jnp.int32),
            pltpu.VMEM((_TAIL, _EMB), jnp.float32),
            pltpu.VMEM((16,), jnp.int32),
        ] + [pltpu.SemaphoreType.DMA] * (2 * _NBUF)
          + [pltpu.VMEM_SHARED((_PAD, _EMB), jnp.float32)],
        compiler_params=pltpu.CompilerParams(use_tc_tiling_on_sc=False,
                                             needs_layout_passes=False),
    )
    return f(emb, src, dst, val)


def _mean_body(a_ref, b_ref, c_ref, d_ref, o_ref):
    o_ref[...] = (a_ref[...] + b_ref[...] + c_ref[...] + d_ref[...]) * 0.25


def _mean4(a, b, c, d):
    blk = (2000, _EMB)
    spec = pl.BlockSpec(blk, lambda i: (i, 0))
    return pl.pallas_call(
        _mean_body,
        grid=(_N_NODES // blk[0],),
        in_specs=[spec] * 4,
        out_specs=spec,
        out_shape=jax.ShapeDtypeStruct((_N_NODES, _EMB), jnp.float32),
    )(a, b, c, d)


def kernel(user_emb, item_emb, edge_src, edge_dst, edge_val):
    e0 = jnp.concatenate([user_emb, item_emb], axis=0)
    e1 = _propagate(e0, edge_src, edge_dst, edge_val)
    e2 = _propagate(e1, edge_src, edge_dst, edge_val)
    e3 = _propagate(e2, edge_src, edge_dst, edge_val)
    m = _mean4(e0, e1, e2, e3)
    return m[:_HALF], m[_HALF:]


# speedup vs baseline: 6.9834x; 1.4952x over previous
import jax
import jax.numpy as jnp
from jax import lax
from jax.experimental import pallas as pl
from jax.experimental.pallas import tpu as pltpu
from jax.experimental.pallas import tpu_sc as plsc

_N_NODES = 100000
_HALF = 50000
_EMB = 32
_E = 1600000
_NC = 2
_NS = 16
_K = 128
_NBUF = 6
_BLK = _K * _NBUF
_EPT = _E // _NS
_NBLK = _EPT // _BLK
_NREM = (_EPT - _NBLK * _BLK) // _K
_TAIL = _EPT - _NBLK * _BLK - _NREM * _K
_PAD = 50176
_TRASH = 50048
_ZSPAN = _PAD // _NS
_CSPAN = 3128
_CLAST = _HALF - 15 * _CSPAN


def _scale_rows(rows_ref, val_ref, zidx_ref, k, vbase=0):
    for g in range(k // 16):
        v16 = val_ref[pl.ds(vbase + g * 16, 16)]
        for t in range(16):
            e = g * 16 + t
            if t == 0:
                idx = zidx_ref[...]
            else:
                idx = jnp.full((16,), t, jnp.int32)
            vs = lax.gather(
                v16, idx[:, None],
                lax.GatherDimensionNumbers(offset_dims=(),
                                           collapsed_slice_dims=(0,),
                                           start_index_map=(0,)),
                slice_sizes=(1,),
                mode=lax.GatherScatterMode.PROMISE_IN_BOUNDS)
            rows_ref[e, pl.ds(0, 16)] = rows_ref[e, pl.ds(0, 16)] * vs
            rows_ref[e, pl.ds(16, 16)] = rows_ref[e, pl.ds(16, 16)] * vs


def _localize_src(srcloc_ref, base, k, src_ref=None, sbase=0):
    rd = srcloc_ref if src_ref is None else src_ref
    for g in range(k // 16):
        s = rd[pl.ds(sbase + g * 16, 16)] - base
        ok = (s >= 0) & (s < _HALF)
        srcloc_ref[pl.ds(g * 16, 16)] = jnp.where(ok, s, _TRASH)


def _prop_body(emb, src, dst, val, out, *scr):
    it = iter(scr)
    dstb, valb, srcb = next(it), next(it), next(it)
    rows = [next(it) for _ in range(_NBUF)]
    sls = [next(it) for _ in range(_NBUF)]
    dst_t, val_t, srcloc_t, rows_t, zidx_v = (
        next(it), next(it), next(it), next(it), next(it))
    gsems = [next(it) for _ in range(_NBUF)]
    ssems = [next(it) for _ in range(_NBUF)]
    acc = next(it)
    rows0 = rows[0]
    sl0 = sls[0]

    cid = lax.axis_index("c")
    sid = lax.axis_index("s")
    base = cid * _HALF
    zidx_v[...] = jnp.zeros((16,), jnp.int32)

    z = jnp.zeros((16,), jnp.float32)

    def zrow(i, carry):
        rows0[i, pl.ds(0, 16)] = z
        rows0[i, pl.ds(16, 16)] = z
        return carry

    lax.fori_loop(0, _K, zrow, 0)

    def zacc(j, carry):
        pltpu.sync_copy(rows0, acc.at[pl.ds(sid * _ZSPAN + j * _K, _K)])
        return carry

    lax.fori_loop(0, _ZSPAN // _K, zacc, 0)
    zrem = _ZSPAN - (_ZSPAN // _K) * _K
    if zrem:
        pltpu.sync_copy(rows0.at[pl.ds(0, zrem)],
                        acc.at[pl.ds(sid * _ZSPAN + (_ZSPAN // _K) * _K, zrem)])
    plsc.subcore_barrier()

    ebase = sid * _EPT

    def block(i, carry):
        off = ebase + i * _BLK
        pltpu.sync_copy(dst.at[pl.ds(off, _BLK)], dstb)
        pltpu.sync_copy(val.at[pl.ds(off, _BLK)], valb)
        pltpu.sync_copy(src.at[pl.ds(off, _BLK)], srcb)
        gd = [pltpu.async_copy(emb.at[dstb.at[pl.ds(j * _K, _K)]],
                               rows[j], gsems[j])
              for j in range(_NBUF)]
        sd = []
        for j in range(_NBUF):
            gd[j].wait()
            _localize_src(sls[j], base, _K, src_ref=srcb, sbase=j * _K)
            _scale_rows(rows[j], valb, zidx_v, _K, vbase=j * _K)
            sd.append(pltpu.async_copy(rows[j], acc.at[sls[j]],
                                       ssems[j], add=True))
        for d in sd:
            d.wait()
        return carry

    lax.fori_loop(0, _NBLK, block, 0)

    for r in range(_NREM):
        roff = ebase + _NBLK * _BLK + r * _K
        pltpu.sync_copy(dst.at[pl.ds(roff, _K)], dstb.at[pl.ds(0, _K)])
        pltpu.sync_copy(val.at[pl.ds(roff, _K)], valb.at[pl.ds(0, _K)])
        pltpu.sync_copy(src.at[pl.ds(roff, _K)], srcb.at[pl.ds(0, _K)])
        pltpu.sync_copy(emb.at[dstb.at[pl.ds(0, _K)]], rows0)
        _localize_src(sl0, base, _K, src_ref=srcb, sbase=0)
        _scale_rows(rows0, valb, zidx_v, _K)
        pltpu.sync_copy(rows0, acc.at[sl0], add=True)

    toff = ebase + _NBLK * _BLK + _NREM * _K
    pltpu.sync_copy(dst.at[pl.ds(toff, _TAIL)], dst_t)
    pltpu.sync_copy(val.at[pl.ds(toff, _TAIL)], val_t)
    pltpu.sync_copy(src.at[pl.ds(toff, _TAIL)], srcloc_t)
    pltpu.sync_copy(emb.at[dst_t], rows_t)
    _localize_src(srcloc_t, base, _TAIL)
    _scale_rows(rows_t, val_t, zidx_v, _TAIL)
    pltpu.sync_copy(rows_t, acc.at[srcloc_t], add=True)

    plsc.subcore_barrier()

    @pl.when(sid < _NS - 1)
    def _copy_main():
        pltpu.sync_copy(acc.at[pl.ds(sid * _CSPAN, _CSPAN)],
                        out.at[pl.ds(base + sid * _CSPAN, _CSPAN)])

    @pl.when(sid == _NS - 1)
    def _copy_last():
        pltpu.sync_copy(acc.at[pl.ds(15 * _CSPAN, _CLAST)],
                        out.at[pl.ds(base + 15 * _CSPAN, _CLAST)])


def _propagate(emb, src, dst, val):
    mesh = plsc.VectorSubcoreMesh(core_axis_name="c", subcore_axis_name="s",
                                  num_cores=_NC, num_subcores=_NS)
    f = pl.kernel(
        _prop_body,
        out_type=jax.ShapeDtypeStruct((_N_NODES, _EMB), jnp.float32),
        mesh=mesh,
        scratch_types=[
            pltpu.VMEM((_BLK,), jnp.int32),
            pltpu.VMEM((_BLK,), jnp.float32),
            pltpu.VMEM((_BLK,), jnp.int32),
        ] + [pltpu.VMEM((_K, _EMB), jnp.float32)] * _NBUF
          + [pltpu.VMEM((_K,), jnp.int32)] * _NBUF
          + [
            pltpu.VMEM((_TAIL,), jnp.int32),
            pltpu.VMEM((_TAIL,), jnp.float32),
            pltpu.VMEM((_TAIL,), jnp.int32),
            pltpu.VMEM((_TAIL, _EMB), jnp.float32),
            pltpu.VMEM((16,), jnp.int32),
        ] + [pltpu.SemaphoreType.DMA] * (2 * _NBUF)
          + [pltpu.VMEM_SHARED((_PAD, _EMB), jnp.float32)],
        compiler_params=pltpu.CompilerParams(use_tc_tiling_on_sc=False,
                                             needs_layout_passes=False),
    )
    return f(emb, src, dst, val)


def _mean_body(a_ref, b_ref, c_ref, d_ref, o_ref):
    o_ref[...] = (a_ref[...] + b_ref[...] + c_ref[...] + d_ref[...]) * 0.25


def _mean4(a, b, c, d):
    blk = (2000, _EMB)
    spec = pl.BlockSpec(blk, lambda i: (i, 0))
    return pl.pallas_call(
        _mean_body,
        grid=(_N_NODES // blk[0],),
        in_specs=[spec] * 4,
        out_specs=spec,
        out_shape=jax.ShapeDtypeStruct((_N_NODES, _EMB), jnp.float32),
    )(a, b, c, d)


def kernel(user_emb, item_emb, edge_src, edge_dst, edge_val):
    e0 = jnp.concatenate([user_emb, item_emb], axis=0)
    e1 = _propagate(e0, edge_src, edge_dst, edge_val)
    e2 = _propagate(e1, edge_src, edge_dst, edge_val)
    e3 = _propagate(e2, edge_src, edge_dst, edge_val)
    m = _mean4(e0, e1, e2, e3)
    return m[:_HALF], m[_HALF:]


# speedup vs baseline: 9.5356x; 1.3655x over previous
import jax
import jax.numpy as jnp
from jax import lax
from jax.experimental import pallas as pl
from jax.experimental.pallas import tpu as pltpu
from jax.experimental.pallas import tpu_sc as plsc

_N_NODES = 100000
_HALF = 50000
_EMB = 32
_E = 1600000
_NC = 2
_NS = 16
_K = 128
_NBUF = 6
_BLK = _K * _NBUF
_PAD = 50176
_TRASH = 50048
_ZSPAN = _PAD // _NS
_CSPAN = 3128
_CLAST = _HALF - 15 * _CSPAN

_NW = _NC * _NS
_PEPW = _E // _NW
_PCH = _PEPW // _K
_PTAIL = _PEPW - _PCH * _K
_RCAP = 50688
_COREOFF = _NW * _RCAP
_PN = 2 * _COREOFF
_SCAP = 3072
_FLUSH = 2048


def _scale_rows(rows_ref, val_ref, zidx_ref, k, vbase=0):
    for g in range(k // 16):
        v16 = val_ref[pl.ds(vbase + g * 16, 16)]
        for t in range(16):
            e = g * 16 + t
            if t == 0:
                idx = zidx_ref[...]
            else:
                idx = jnp.full((16,), t, jnp.int32)
            vs = lax.gather(
                v16, idx[:, None],
                lax.GatherDimensionNumbers(offset_dims=(),
                                           collapsed_slice_dims=(0,),
                                           start_index_map=(0,)),
                slice_sizes=(1,),
                mode=lax.GatherScatterMode.PROMISE_IN_BOUNDS)
            rows_ref[e, pl.ds(0, 16)] = rows_ref[e, pl.ds(0, 16)] * vs
            rows_ref[e, pl.ds(16, 16)] = rows_ref[e, pl.ds(16, 16)] * vs


def _localize_src(srcloc_ref, base, k, src_ref, sbase=0):
    for g in range(k // 16):
        s = src_ref[pl.ds(sbase + g * 16, 16)] - base
        ok = (s >= 0) & (s < _HALF)
        srcloc_ref[pl.ds(g * 16, 16)] = jnp.where(ok, s, _TRASH)


def _part_body(src, dst, val, srcp, dstp, valp, counts, *scr):
    it = iter(scr)
    inb_s, inb_d, inb_v = next(it), next(it), next(it)
    st_s = [next(it), next(it)]
    st_d = [next(it), next(it)]
    st_v = [next(it), next(it)]
    cst = next(it)

    cid = lax.axis_index("c")
    sid = lax.axis_index("s")
    wid = cid * _NS + sid
    ebase = wid * _PEPW
    hbase = [wid * _RCAP, _COREOFF + wid * _RCAP]

    lanes = jnp.arange(16, dtype=jnp.int32)

    def groups(fill0, fill1, n16):
        fills = [fill0, fill1]
        for g in range(n16):
            s16 = inb_s[pl.ds(g * 16, 16)]
            d16 = inb_d[pl.ds(g * 16, 16)]
            v16 = inb_v[pl.ds(g * 16, 16)]
            m0 = s16 < _HALF
            c0 = jnp.sum(m0.astype(jnp.int32))
            for p in range(2):
                m = m0 if p == 0 else jnp.logical_not(m0)
                plsc.store_compressed(st_s[p].at[pl.ds(fills[p], 16)], s16,
                                      mask=m)
                plsc.store_compressed(st_d[p].at[pl.ds(fills[p], 16)], d16,
                                      mask=m)
                plsc.store_compressed(st_v[p].at[pl.ds(fills[p], 16)], v16,
                                      mask=m)
            fills[0] = fills[0] + c0
            fills[1] = fills[1] + (16 - c0)
        return fills[0], fills[1]

    def maybe_flush(p, fill, off):
        do = fill >= _FLUSH

        @pl.when(do)
        def _():
            o = pl.multiple_of(hbase[p] + off, _FLUSH)
            pltpu.sync_copy(st_s[p].at[pl.ds(0, _FLUSH)],
                            srcp.at[pl.ds(o, _FLUSH)])
            pltpu.sync_copy(st_d[p].at[pl.ds(0, _FLUSH)],
                            dstp.at[pl.ds(o, _FLUSH)])
            pltpu.sync_copy(st_v[p].at[pl.ds(0, _FLUSH)],
                            valp.at[pl.ds(o, _FLUSH)])
            for g in range(8):
                st_s[p][pl.ds(g * 16, 16)] = st_s[p][pl.ds(_FLUSH + g * 16, 16)]
                st_d[p][pl.ds(g * 16, 16)] = st_d[p][pl.ds(_FLUSH + g * 16, 16)]
                st_v[p][pl.ds(g * 16, 16)] = st_v[p][pl.ds(_FLUSH + g * 16, 16)]

        fill = jnp.where(do, fill - _FLUSH, fill)
        off = jnp.where(do, off + _FLUSH, off)
        return fill, off

    def chunk(i, carry):
        fill0, off0, fill1, off1 = carry
        off = ebase + i * _K
        pltpu.sync_copy(src.at[pl.ds(off, _K)], inb_s)
        pltpu.sync_copy(dst.at[pl.ds(off, _K)], inb_d)
        pltpu.sync_copy(val.at[pl.ds(off, _K)], inb_v)
        fill0, fill1 = groups(fill0, fill1, _K // 16)
        fill0, off0 = maybe_flush(0, fill0, off0)
        fill1, off1 = maybe_flush(1, fill1, off1)
        return fill0, off0, fill1, off1

    z = jnp.int32(0)
    fill0, off0, fill1, off1 = lax.fori_loop(0, _PCH, chunk, (z, z, z, z))

    toff = ebase + _PCH * _K
    pltpu.sync_copy(src.at[pl.ds(toff, _PTAIL)], inb_s.at[pl.ds(0, _PTAIL)])
    pltpu.sync_copy(dst.at[pl.ds(toff, _PTAIL)], inb_d.at[pl.ds(0, _PTAIL)])
    pltpu.sync_copy(val.at[pl.ds(toff, _PTAIL)], inb_v.at[pl.ds(0, _PTAIL)])
    fill0, fill1 = groups(fill0, fill1, _PTAIL // 16)

    fills = [fill0, fill1]
    offs = [off0, off1]
    for p in range(2):
        fill, off = fills[p], offs[p]
        total = off + fill
        padded = ((total + _BLK - 1) // _BLK) * _BLK
        stage_end = padded - off
        b16 = (fill // 16) * 16
        for g in range(50):
            og = b16 + g * 16
            lane = og + lanes
            keep = lane < fill
            st_s[p][pl.ds(og, 16)] = jnp.where(keep, st_s[p][pl.ds(og, 16)], 0)
            st_d[p][pl.ds(og, 16)] = jnp.where(keep, st_d[p][pl.ds(og, 16)], 0)
            st_v[p][pl.ds(og, 16)] = jnp.where(
                keep, st_v[p][pl.ds(og, 16)], 0.0)
        ndrain = stage_end // _K

        def drain(j, carry, _p=p, _off=off):
            o = pl.multiple_of(hbase[_p] + _off + j * _K, _K)
            s = pl.multiple_of(j * _K, _K)
            pltpu.sync_copy(st_s[_p].at[pl.ds(s, _K)], srcp.at[pl.ds(o, _K)])
            pltpu.sync_copy(st_d[_p].at[pl.ds(s, _K)], dstp.at[pl.ds(o, _K)])
            pltpu.sync_copy(st_v[_p].at[pl.ds(s, _K)], valp.at[pl.ds(o, _K)])
            return carry

        lax.fori_loop(0, ndrain, drain, 0)
        nblk = padded // _BLK
        cst[...] = jnp.full((16,), 1, jnp.int32) * nblk
        row = pl.multiple_of((p * _NW + wid) * 16, 16)
        pltpu.sync_copy(cst, counts.at[pl.ds(row, 16)])


def _partition(src, dst, val):
    mesh = plsc.VectorSubcoreMesh(core_axis_name="c", subcore_axis_name="s",
                                  num_cores=_NC, num_subcores=_NS)
    f = pl.kernel(
        _part_body,
        out_type=(
            jax.ShapeDtypeStruct((_PN,), jnp.int32),
            jax.ShapeDtypeStruct((_PN,), jnp.int32),
            jax.ShapeDtypeStruct((_PN,), jnp.float32),
            jax.ShapeDtypeStruct((2 * _NW * 16,), jnp.int32),
        ),
        mesh=mesh,
        scratch_types=[
            pltpu.VMEM((_K,), jnp.int32),
            pltpu.VMEM((_K,), jnp.int32),
            pltpu.VMEM((_K,), jnp.float32),
            pltpu.VMEM((_SCAP,), jnp.int32),
            pltpu.VMEM((_SCAP,), jnp.int32),
            pltpu.VMEM((_SCAP,), jnp.int32),
            pltpu.VMEM((_SCAP,), jnp.int32),
            pltpu.VMEM((_SCAP,), jnp.float32),
            pltpu.VMEM((_SCAP,), jnp.float32),
            pltpu.VMEM((16,), jnp.int32),
        ],
        compiler_params=pltpu.CompilerParams(use_tc_tiling_on_sc=False,
                                             needs_layout_passes=False),
    )
    return f(src, dst, val)


def _prop_body(emb, srcp, dstp, valp, counts, out, *scr):
    it = iter(scr)
    dstb, valb, srcb = next(it), next(it), next(it)
    rows = [next(it) for _ in range(_NBUF)]
    sls = [next(it) for _ in range(_NBUF)]
    zidx_v, cnts_v = next(it), next(it)
    gsems = [next(it) for _ in range(_NBUF)]
    ssems = [next(it) for _ in range(_NBUF)]
    acc = next(it)
    rows0 = rows[0]

    cid = lax.axis_index("c")
    sid = lax.axis_index("s")
    base = cid * _HALF
    zidx_v[...] = jnp.zeros((16,), jnp.int32)

    z = jnp.zeros((16,), jnp.float32)

    def zrow(i, carry):
        rows0[i, pl.ds(0, 16)] = z
        rows0[i, pl.ds(16, 16)] = z
        return carry

    lax.fori_loop(0, _K, zrow, 0)

    def zacc(j, carry):
        pltpu.sync_copy(rows0, acc.at[pl.ds(sid * _ZSPAN + j * _K, _K)])
        return carry

    lax.fori_loop(0, _ZSPAN // _K, zacc, 0)
    zrem = _ZSPAN - (_ZSPAN // _K) * _K
    if zrem:
        pltpu.sync_copy(rows0.at[pl.ds(0, zrem)],
                        acc.at[pl.ds(sid * _ZSPAN + (_ZSPAN // _K) * _K, zrem)])
    plsc.subcore_barrier()

    def region(rr, carry):
        w = sid * 2 + rr
        row = pl.multiple_of((cid * _NW + w) * 16, 16)
        pltpu.sync_copy(counts.at[pl.ds(row, 16)], cnts_v)
        nblk = cnts_v[...][0]
        rbase = cid * _COREOFF + w * _RCAP

        def block(i, c2):
            off = pl.multiple_of(rbase + i * _BLK, _BLK)
            pltpu.sync_copy(dstp.at[pl.ds(off, _BLK)], dstb)
            pltpu.sync_copy(valp.at[pl.ds(off, _BLK)], valb)
            pltpu.sync_copy(srcp.at[pl.ds(off, _BLK)], srcb)
            gd = [pltpu.async_copy(emb.at[dstb.at[pl.ds(j * _K, _K)]],
                                   rows[j], gsems[j])
                  for j in range(_NBUF)]
            sd = []
            for j in range(_NBUF):
                gd[j].wait()
                _localize_src(sls[j], base, _K, src_ref=srcb, sbase=j * _K)
                _scale_rows(rows[j], valb, zidx_v, _K, vbase=j * _K)
                sd.append(pltpu.async_copy(rows[j], acc.at[sls[j]],
                                           ssems[j], add=True))
            for d in sd:
                d.wait()
            return c2

        lax.fori_loop(0, nblk, block, 0)
        return carry

    lax.fori_loop(0, 2, region, 0)

    plsc.subcore_barrier()

    @pl.when(sid < _NS - 1)
    def _copy_main():
        pltpu.sync_copy(acc.at[pl.ds(sid * _CSPAN, _CSPAN)],
                        out.at[pl.ds(base + sid * _CSPAN, _CSPAN)])

    @pl.when(sid == _NS - 1)
    def _copy_last():
        pltpu.sync_copy(acc.at[pl.ds(15 * _CSPAN, _CLAST)],
                        out.at[pl.ds(base + 15 * _CSPAN, _CLAST)])


def _propagate(emb, srcp, dstp, valp, counts):
    mesh = plsc.VectorSubcoreMesh(core_axis_name="c", subcore_axis_name="s",
                                  num_cores=_NC, num_subcores=_NS)
    f = pl.kernel(
        _prop_body,
        out_type=jax.ShapeDtypeStruct((_N_NODES, _EMB), jnp.float32),
        mesh=mesh,
        scratch_types=[
            pltpu.VMEM((_BLK,), jnp.int32),
            pltpu.VMEM((_BLK,), jnp.float32),
            pltpu.VMEM((_BLK,), jnp.int32),
        ] + [pltpu.VMEM((_K, _EMB), jnp.float32)] * _NBUF
          + [pltpu.VMEM((_K,), jnp.int32)] * _NBUF
          + [
            pltpu.VMEM((16,), jnp.int32),
            pltpu.VMEM((16,), jnp.int32),
        ] + [pltpu.SemaphoreType.DMA] * (2 * _NBUF)
          + [pltpu.VMEM_SHARED((_PAD, _EMB), jnp.float32)],
        compiler_params=pltpu.CompilerParams(use_tc_tiling_on_sc=False,
                                             needs_layout_passes=False),
    )
    return f(emb, srcp, dstp, valp, counts)


def _mean_body(a_ref, b_ref, c_ref, d_ref, o_ref):
    o_ref[...] = (a_ref[...] + b_ref[...] + c_ref[...] + d_ref[...]) * 0.25


def _mean4(a, b, c, d):
    blk = (2000, _EMB)
    spec = pl.BlockSpec(blk, lambda i: (i, 0))
    return pl.pallas_call(
        _mean_body,
        grid=(_N_NODES // blk[0],),
        in_specs=[spec] * 4,
        out_specs=spec,
        out_shape=jax.ShapeDtypeStruct((_N_NODES, _EMB), jnp.float32),
    )(a, b, c, d)


def kernel(user_emb, item_emb, edge_src, edge_dst, edge_val):
    e0 = jnp.concatenate([user_emb, item_emb], axis=0)
    srcp, dstp, valp, counts = _partition(edge_src, edge_dst, edge_val)
    e1 = _propagate(e0, srcp, dstp, valp, counts)
    e2 = _propagate(e1, srcp, dstp, valp, counts)
    e3 = _propagate(e2, srcp, dstp, valp, counts)
    m = _mean4(e0, e1, e2, e3)
    return m[:_HALF], m[_HALF:]


# speedup vs baseline: 10.2592x; 1.0759x over previous
import jax
import jax.numpy as jnp
from jax import lax
from jax.experimental import pallas as pl
from jax.experimental.pallas import tpu as pltpu
from jax.experimental.pallas import tpu_sc as plsc

_N_NODES = 100000
_HALF = 50000
_EMB = 32
_E = 1600000
_NC = 2
_NS = 16
_K = 128
_NBUF = 6
_BLK = _K * _NBUF
_PAD = 50176
_TRASH = 50048
_ZSPAN = _PAD // _NS
_CSPAN = 3128
_CLAST = _HALF - 15 * _CSPAN

_NW = _NC * _NS
_PEPW = _E // _NW
_PCH = _PEPW // _K
_PTAIL = _PEPW - _PCH * _K
_RCAP = 50688
_COREOFF = _NW * _RCAP
_PN = 2 * _COREOFF
_SCAP = 3072
_FLUSH = 2048


def _scale_rows(rows_ref, val_ref, zidx_ref, k, vbase=0):
    for g in range(k // 16):
        v16 = val_ref[pl.ds(vbase + g * 16, 16)]
        for t in range(16):
            e = g * 16 + t
            if t == 0:
                idx = zidx_ref[...]
            else:
                idx = jnp.full((16,), t, jnp.int32)
            vs = lax.gather(
                v16, idx[:, None],
                lax.GatherDimensionNumbers(offset_dims=(),
                                           collapsed_slice_dims=(0,),
                                           start_index_map=(0,)),
                slice_sizes=(1,),
                mode=lax.GatherScatterMode.PROMISE_IN_BOUNDS)
            rows_ref[e, pl.ds(0, 16)] = rows_ref[e, pl.ds(0, 16)] * vs
            rows_ref[e, pl.ds(16, 16)] = rows_ref[e, pl.ds(16, 16)] * vs


def _localize_src(srcloc_ref, base, k, src_ref, sbase=0):
    for g in range(k // 16):
        s = src_ref[pl.ds(sbase + g * 16, 16)] - base
        ok = (s >= 0) & (s < _HALF)
        srcloc_ref[pl.ds(g * 16, 16)] = jnp.where(ok, s, _TRASH)


def _part_body(src, dst, val, srcp, dstp, valp, counts, *scr):
    it = iter(scr)
    inb_s, inb_d, inb_v = next(it), next(it), next(it)
    st_s = [next(it), next(it)]
    st_d = [next(it), next(it)]
    st_v = [next(it), next(it)]
    cst = next(it)

    cid = lax.axis_index("c")
    sid = lax.axis_index("s")
    wid = cid * _NS + sid
    ebase = wid * _PEPW
    hbase = [wid * _RCAP, _COREOFF + wid * _RCAP]

    lanes = jnp.arange(16, dtype=jnp.int32)

    def groups(fill0, fill1, n16):
        fills = [fill0, fill1]
        for g in range(n16):
            s16 = inb_s[pl.ds(g * 16, 16)]
            d16 = inb_d[pl.ds(g * 16, 16)]
            v16 = inb_v[pl.ds(g * 16, 16)]
            m0 = s16 < _HALF
            c0 = jnp.sum(m0.astype(jnp.int32))
            for p in range(2):
                m = m0 if p == 0 else jnp.logical_not(m0)
                plsc.store_compressed(st_s[p].at[pl.ds(fills[p], 16)], s16,
                                      mask=m)
                plsc.store_compressed(st_d[p].at[pl.ds(fills[p], 16)], d16,
                                      mask=m)
                plsc.store_compressed(st_v[p].at[pl.ds(fills[p], 16)], v16,
                                      mask=m)
            fills[0] = fills[0] + c0
            fills[1] = fills[1] + (16 - c0)
        return fills[0], fills[1]

    def maybe_flush(p, fill, off):
        do = fill >= _FLUSH

        @pl.when(do)
        def _():
            o = pl.multiple_of(hbase[p] + off, _FLUSH)
            pltpu.sync_copy(st_s[p].at[pl.ds(0, _FLUSH)],
                            srcp.at[pl.ds(o, _FLUSH)])
            pltpu.sync_copy(st_d[p].at[pl.ds(0, _FLUSH)],
                            dstp.at[pl.ds(o, _FLUSH)])
            pltpu.sync_copy(st_v[p].at[pl.ds(0, _FLUSH)],
                            valp.at[pl.ds(o, _FLUSH)])
            for g in range(8):
                st_s[p][pl.ds(g * 16, 16)] = st_s[p][pl.ds(_FLUSH + g * 16, 16)]
                st_d[p][pl.ds(g * 16, 16)] = st_d[p][pl.ds(_FLUSH + g * 16, 16)]
                st_v[p][pl.ds(g * 16, 16)] = st_v[p][pl.ds(_FLUSH + g * 16, 16)]

        fill = jnp.where(do, fill - _FLUSH, fill)
        off = jnp.where(do, off + _FLUSH, off)
        return fill, off

    def chunk(i, carry):
        fill0, off0, fill1, off1 = carry
        off = ebase + i * _K
        pltpu.sync_copy(src.at[pl.ds(off, _K)], inb_s)
        pltpu.sync_copy(dst.at[pl.ds(off, _K)], inb_d)
        pltpu.sync_copy(val.at[pl.ds(off, _K)], inb_v)
        fill0, fill1 = groups(fill0, fill1, _K // 16)
        fill0, off0 = maybe_flush(0, fill0, off0)
        fill1, off1 = maybe_flush(1, fill1, off1)
        return fill0, off0, fill1, off1

    z = jnp.int32(0)
    fill0, off0, fill1, off1 = lax.fori_loop(0, _PCH, chunk, (z, z, z, z))

    toff = ebase + _PCH * _K
    pltpu.sync_copy(src.at[pl.ds(toff, _PTAIL)], inb_s.at[pl.ds(0, _PTAIL)])
    pltpu.sync_copy(dst.at[pl.ds(toff, _PTAIL)], inb_d.at[pl.ds(0, _PTAIL)])
    pltpu.sync_copy(val.at[pl.ds(toff, _PTAIL)], inb_v.at[pl.ds(0, _PTAIL)])
    fill0, fill1 = groups(fill0, fill1, _PTAIL // 16)

    fills = [fill0, fill1]
    offs = [off0, off1]
    for p in range(2):
        fill, off = fills[p], offs[p]
        total = off + fill
        padded = ((total + _BLK - 1) // _BLK) * _BLK
        stage_end = padded - off
        b16 = (fill // 16) * 16
        for g in range(50):
            og = b16 + g * 16
            lane = og + lanes
            keep = lane < fill
            st_s[p][pl.ds(og, 16)] = jnp.where(keep, st_s[p][pl.ds(og, 16)], 0)
            st_d[p][pl.ds(og, 16)] = jnp.where(keep, st_d[p][pl.ds(og, 16)], 0)
            st_v[p][pl.ds(og, 16)] = jnp.where(
                keep, st_v[p][pl.ds(og, 16)], 0.0)
        ndrain = stage_end // _K

        def drain(j, carry, _p=p, _off=off):
            o = pl.multiple_of(hbase[_p] + _off + j * _K, _K)
            s = pl.multiple_of(j * _K, _K)
            pltpu.sync_copy(st_s[_p].at[pl.ds(s, _K)], srcp.at[pl.ds(o, _K)])
            pltpu.sync_copy(st_d[_p].at[pl.ds(s, _K)], dstp.at[pl.ds(o, _K)])
            pltpu.sync_copy(st_v[_p].at[pl.ds(s, _K)], valp.at[pl.ds(o, _K)])
            return carry

        lax.fori_loop(0, ndrain, drain, 0)
        nblk = padded // _BLK
        cst[...] = jnp.full((16,), 1, jnp.int32) * nblk
        row = pl.multiple_of((p * _NW + wid) * 16, 16)
        pltpu.sync_copy(cst, counts.at[pl.ds(row, 16)])


def _partition(src, dst, val):
    mesh = plsc.VectorSubcoreMesh(core_axis_name="c", subcore_axis_name="s",
                                  num_cores=_NC, num_subcores=_NS)
    f = pl.kernel(
        _part_body,
        out_type=(
            jax.ShapeDtypeStruct((_PN,), jnp.int32),
            jax.ShapeDtypeStruct((_PN,), jnp.int32),
            jax.ShapeDtypeStruct((_PN,), jnp.float32),
            jax.ShapeDtypeStruct((2 * _NW * 16,), jnp.int32),
        ),
        mesh=mesh,
        scratch_types=[
            pltpu.VMEM((_K,), jnp.int32),
            pltpu.VMEM((_K,), jnp.int32),
            pltpu.VMEM((_K,), jnp.float32),
            pltpu.VMEM((_SCAP,), jnp.int32),
            pltpu.VMEM((_SCAP,), jnp.int32),
            pltpu.VMEM((_SCAP,), jnp.int32),
            pltpu.VMEM((_SCAP,), jnp.int32),
            pltpu.VMEM((_SCAP,), jnp.float32),
            pltpu.VMEM((_SCAP,), jnp.float32),
            pltpu.VMEM((16,), jnp.int32),
        ],
        compiler_params=pltpu.CompilerParams(use_tc_tiling_on_sc=False,
                                             needs_layout_passes=False),
    )
    return f(src, dst, val)


def _prop_body(emb, srcp, dstp, valp, counts, out, *scr):
    it = iter(scr)
    dstb, valb, srcb = next(it), next(it), next(it)
    rows = [next(it) for _ in range(_NBUF)]
    sls = [next(it) for _ in range(_NBUF)]
    zidx_v, cnts_v = next(it), next(it)
    gsems = [next(it) for _ in range(_NBUF)]
    ssems = [next(it) for _ in range(_NBUF)]
    msem = next(it)
    acc = next(it)
    rows0 = rows[0]

    cid = lax.axis_index("c")
    sid = lax.axis_index("s")
    base = cid * _HALF
    zidx_v[...] = jnp.zeros((16,), jnp.int32)

    z = jnp.zeros((16,), jnp.float32)

    def zrow(i, carry):
        rows0[i, pl.ds(0, 16)] = z
        rows0[i, pl.ds(16, 16)] = z
        return carry

    lax.fori_loop(0, _K, zrow, 0)

    def zacc(j, carry):
        pltpu.sync_copy(rows0, acc.at[pl.ds(sid * _ZSPAN + j * _K, _K)])
        return carry

    lax.fori_loop(0, _ZSPAN // _K, zacc, 0)
    zrem = _ZSPAN - (_ZSPAN // _K) * _K
    if zrem:
        pltpu.sync_copy(rows0.at[pl.ds(0, zrem)],
                        acc.at[pl.ds(sid * _ZSPAN + (_ZSPAN // _K) * _K, zrem)])
    plsc.subcore_barrier()

    def region(rr, carry):
        w = sid * 2 + rr
        row = pl.multiple_of((cid * _NW + w) * 16, 16)
        pltpu.sync_copy(counts.at[pl.ds(row, 16)], cnts_v)
        nblk = cnts_v[...][0]
        rbase = cid * _COREOFF + w * _RCAP

        def block(i, c2):
            off = pl.multiple_of(rbase + i * _BLK, _BLK)
            md = [pltpu.async_copy(dstp.at[pl.ds(off, _BLK)], dstb, msem),
                  pltpu.async_copy(valp.at[pl.ds(off, _BLK)], valb, msem),
                  pltpu.async_copy(srcp.at[pl.ds(off, _BLK)], srcb, msem)]
            for d in md:
                d.wait()
            gd = [pltpu.async_copy(emb.at[dstb.at[pl.ds(j * _K, _K)]],
                                   rows[j], gsems[j])
                  for j in range(_NBUF)]
            sd = []
            for j in range(_NBUF):
                gd[j].wait()
                _localize_src(sls[j], base, _K, src_ref=srcb, sbase=j * _K)
                _scale_rows(rows[j], valb, zidx_v, _K, vbase=j * _K)
                sd.append(pltpu.async_copy(rows[j], acc.at[sls[j]],
                                           ssems[j], add=True))
            for d in sd:
                d.wait()
            return c2

        lax.fori_loop(0, nblk, block, 0)
        return carry

    lax.fori_loop(0, 2, region, 0)

    plsc.subcore_barrier()

    @pl.when(sid < _NS - 1)
    def _copy_main():
        pltpu.sync_copy(acc.at[pl.ds(sid * _CSPAN, _CSPAN)],
                        out.at[pl.ds(base + sid * _CSPAN, _CSPAN)])

    @pl.when(sid == _NS - 1)
    def _copy_last():
        pltpu.sync_copy(acc.at[pl.ds(15 * _CSPAN, _CLAST)],
                        out.at[pl.ds(base + 15 * _CSPAN, _CLAST)])


def _propagate(emb, srcp, dstp, valp, counts):
    mesh = plsc.VectorSubcoreMesh(core_axis_name="c", subcore_axis_name="s",
                                  num_cores=_NC, num_subcores=_NS)
    f = pl.kernel(
        _prop_body,
        out_type=jax.ShapeDtypeStruct((_N_NODES, _EMB), jnp.float32),
        mesh=mesh,
        scratch_types=[
            pltpu.VMEM((_BLK,), jnp.int32),
            pltpu.VMEM((_BLK,), jnp.float32),
            pltpu.VMEM((_BLK,), jnp.int32),
        ] + [pltpu.VMEM((_K, _EMB), jnp.float32)] * _NBUF
          + [pltpu.VMEM((_K,), jnp.int32)] * _NBUF
          + [
            pltpu.VMEM((16,), jnp.int32),
            pltpu.VMEM((16,), jnp.int32),
        ] + [pltpu.SemaphoreType.DMA] * (2 * _NBUF + 1)
          + [pltpu.VMEM_SHARED((_PAD, _EMB), jnp.float32)],
        compiler_params=pltpu.CompilerParams(use_tc_tiling_on_sc=False,
                                             needs_layout_passes=False),
    )
    return f(emb, srcp, dstp, valp, counts)


def _mean_body(a_ref, b_ref, c_ref, d_ref, o_ref):
    o_ref[...] = (a_ref[...] + b_ref[...] + c_ref[...] + d_ref[...]) * 0.25


def _mean4(a, b, c, d):
    blk = (2000, _EMB)
    spec = pl.BlockSpec(blk, lambda i: (i, 0))
    return pl.pallas_call(
        _mean_body,
        grid=(_N_NODES // blk[0],),
        in_specs=[spec] * 4,
        out_specs=spec,
        out_shape=jax.ShapeDtypeStruct((_N_NODES, _EMB), jnp.float32),
    )(a, b, c, d)


def kernel(user_emb, item_emb, edge_src, edge_dst, edge_val):
    e0 = jnp.concatenate([user_emb, item_emb], axis=0)
    srcp, dstp, valp, counts = _partition(edge_src, edge_dst, edge_val)
    e1 = _propagate(e0, srcp, dstp, valp, counts)
    e2 = _propagate(e1, srcp, dstp, valp, counts)
    e3 = _propagate(e2, srcp, dstp, valp, counts)
    m = _mean4(e0, e1, e2, e3)
    return m[:_HALF], m[_HALF:]


# speedup vs baseline: 10.2786x; 1.0019x over previous
import jax
import jax.numpy as jnp
from jax import lax
from jax.experimental import pallas as pl
from jax.experimental.pallas import tpu as pltpu
from jax.experimental.pallas import tpu_sc as plsc

_N_NODES = 100000
_HALF = 50000
_EMB = 32
_E = 1600000
_NC = 2
_NS = 16
_K = 128
_NBUF = 6
_BLK = _K * _NBUF
_PAD = 50176
_TRASH = 50048
_ZSPAN = _PAD // _NS
_CSPAN = 3128
_CLAST = _HALF - 15 * _CSPAN

_NW = _NC * _NS
_PEPW = _E // _NW
_PCH = _PEPW // _K
_PTAIL = _PEPW - _PCH * _K
_RCAP = 50688
_COREOFF = _NW * _RCAP
_PN = 2 * _COREOFF
_SCAP = 3072
_FLUSH = 2048


def _scale_rows(rows_ref, val_ref, zidx_ref, k, vbase=0):
    for g in range(k // 16):
        v16 = val_ref[pl.ds(vbase + g * 16, 16)]
        for t in range(16):
            e = g * 16 + t
            if t == 0:
                idx = zidx_ref[...]
            else:
                idx = jnp.full((16,), t, jnp.int32)
            vs = lax.gather(
                v16, idx[:, None],
                lax.GatherDimensionNumbers(offset_dims=(),
                                           collapsed_slice_dims=(0,),
                                           start_index_map=(0,)),
                slice_sizes=(1,),
                mode=lax.GatherScatterMode.PROMISE_IN_BOUNDS)
            rows_ref[e, pl.ds(0, 16)] = rows_ref[e, pl.ds(0, 16)] * vs
            rows_ref[e, pl.ds(16, 16)] = rows_ref[e, pl.ds(16, 16)] * vs


def _part_body(src, dst, val, srcp, dstp, valp, counts, *scr):
    it = iter(scr)
    inb_s, inb_d, inb_v = next(it), next(it), next(it)
    st_s = [next(it), next(it)]
    st_d = [next(it), next(it)]
    st_v = [next(it), next(it)]
    cst = next(it)

    cid = lax.axis_index("c")
    sid = lax.axis_index("s")
    wid = cid * _NS + sid
    ebase = wid * _PEPW
    hbase = [wid * _RCAP, _COREOFF + wid * _RCAP]

    lanes = jnp.arange(16, dtype=jnp.int32)

    def groups(fill0, fill1, n16):
        fills = [fill0, fill1]
        for g in range(n16):
            s16 = inb_s[pl.ds(g * 16, 16)]
            d16 = inb_d[pl.ds(g * 16, 16)]
            v16 = inb_v[pl.ds(g * 16, 16)]
            m0 = s16 < _HALF
            c0 = jnp.sum(m0.astype(jnp.int32))
            for p in range(2):
                m = m0 if p == 0 else jnp.logical_not(m0)
                sloc = s16 if p == 0 else s16 - _HALF
                plsc.store_compressed(st_s[p].at[pl.ds(fills[p], 16)], sloc,
                                      mask=m)
                plsc.store_compressed(st_d[p].at[pl.ds(fills[p], 16)], d16,
                                      mask=m)
                plsc.store_compressed(st_v[p].at[pl.ds(fills[p], 16)], v16,
                                      mask=m)
            fills[0] = fills[0] + c0
            fills[1] = fills[1] + (16 - c0)
        return fills[0], fills[1]

    def maybe_flush(p, fill, off):
        do = fill >= _FLUSH

        @pl.when(do)
        def _():
            o = pl.multiple_of(hbase[p] + off, _FLUSH)
            pltpu.sync_copy(st_s[p].at[pl.ds(0, _FLUSH)],
                            srcp.at[pl.ds(o, _FLUSH)])
            pltpu.sync_copy(st_d[p].at[pl.ds(0, _FLUSH)],
                            dstp.at[pl.ds(o, _FLUSH)])
            pltpu.sync_copy(st_v[p].at[pl.ds(0, _FLUSH)],
                            valp.at[pl.ds(o, _FLUSH)])
            for g in range(8):
                st_s[p][pl.ds(g * 16, 16)] = st_s[p][pl.ds(_FLUSH + g * 16, 16)]
                st_d[p][pl.ds(g * 16, 16)] = st_d[p][pl.ds(_FLUSH + g * 16, 16)]
                st_v[p][pl.ds(g * 16, 16)] = st_v[p][pl.ds(_FLUSH + g * 16, 16)]

        fill = jnp.where(do, fill - _FLUSH, fill)
        off = jnp.where(do, off + _FLUSH, off)
        return fill, off

    def chunk(i, carry):
        fill0, off0, fill1, off1 = carry
        off = ebase + i * _K
        pltpu.sync_copy(src.at[pl.ds(off, _K)], inb_s)
        pltpu.sync_copy(dst.at[pl.ds(off, _K)], inb_d)
        pltpu.sync_copy(val.at[pl.ds(off, _K)], inb_v)
        fill0, fill1 = groups(fill0, fill1, _K // 16)
        fill0, off0 = maybe_flush(0, fill0, off0)
        fill1, off1 = maybe_flush(1, fill1, off1)
        return fill0, off0, fill1, off1

    z = jnp.int32(0)
    fill0, off0, fill1, off1 = lax.fori_loop(0, _PCH, chunk, (z, z, z, z))

    toff = ebase + _PCH * _K
    pltpu.sync_copy(src.at[pl.ds(toff, _PTAIL)], inb_s.at[pl.ds(0, _PTAIL)])
    pltpu.sync_copy(dst.at[pl.ds(toff, _PTAIL)], inb_d.at[pl.ds(0, _PTAIL)])
    pltpu.sync_copy(val.at[pl.ds(toff, _PTAIL)], inb_v.at[pl.ds(0, _PTAIL)])
    fill0, fill1 = groups(fill0, fill1, _PTAIL // 16)

    fills = [fill0, fill1]
    offs = [off0, off1]
    for p in range(2):
        fill, off = fills[p], offs[p]
        total = off + fill
        padded = ((total + _BLK - 1) // _BLK) * _BLK
        stage_end = padded - off
        b16 = (fill // 16) * 16
        for g in range(50):
            og = b16 + g * 16
            lane = og + lanes
            keep = lane < fill
            st_s[p][pl.ds(og, 16)] = jnp.where(keep, st_s[p][pl.ds(og, 16)], 0)
            st_d[p][pl.ds(og, 16)] = jnp.where(keep, st_d[p][pl.ds(og, 16)], 0)
            st_v[p][pl.ds(og, 16)] = jnp.where(
                keep, st_v[p][pl.ds(og, 16)], 0.0)
        ndrain = stage_end // _K

        def drain(j, carry, _p=p, _off=off):
            o = pl.multiple_of(hbase[_p] + _off + j * _K, _K)
            s = pl.multiple_of(j * _K, _K)
            pltpu.sync_copy(st_s[_p].at[pl.ds(s, _K)], srcp.at[pl.ds(o, _K)])
            pltpu.sync_copy(st_d[_p].at[pl.ds(s, _K)], dstp.at[pl.ds(o, _K)])
            pltpu.sync_copy(st_v[_p].at[pl.ds(s, _K)], valp.at[pl.ds(o, _K)])
            return carry

        lax.fori_loop(0, ndrain, drain, 0)
        nblk = padded // _BLK
        cst[...] = jnp.full((16,), 1, jnp.int32) * nblk
        row = pl.multiple_of((p * _NW + wid) * 16, 16)
        pltpu.sync_copy(cst, counts.at[pl.ds(row, 16)])


def _partition(src, dst, val):
    mesh = plsc.VectorSubcoreMesh(core_axis_name="c", subcore_axis_name="s",
                                  num_cores=_NC, num_subcores=_NS)
    f = pl.kernel(
        _part_body,
        out_type=(
            jax.ShapeDtypeStruct((_PN,), jnp.int32),
            jax.ShapeDtypeStruct((_PN,), jnp.int32),
            jax.ShapeDtypeStruct((_PN,), jnp.float32),
            jax.ShapeDtypeStruct((2 * _NW * 16,), jnp.int32),
        ),
        mesh=mesh,
        scratch_types=[
            pltpu.VMEM((_K,), jnp.int32),
            pltpu.VMEM((_K,), jnp.int32),
            pltpu.VMEM((_K,), jnp.float32),
            pltpu.VMEM((_SCAP,), jnp.int32),
            pltpu.VMEM((_SCAP,), jnp.int32),
            pltpu.VMEM((_SCAP,), jnp.int32),
            pltpu.VMEM((_SCAP,), jnp.int32),
            pltpu.VMEM((_SCAP,), jnp.float32),
            pltpu.VMEM((_SCAP,), jnp.float32),
            pltpu.VMEM((16,), jnp.int32),
        ],
        compiler_params=pltpu.CompilerParams(use_tc_tiling_on_sc=False,
                                             needs_layout_passes=False),
    )
    return f(src, dst, val)


def _prop_body(emb, srcp, dstp, valp, counts, out, *scr):
    it = iter(scr)
    dstb, valb, srcb = next(it), next(it), next(it)
    rows = [next(it) for _ in range(_NBUF)]
    sls = [next(it) for _ in range(_NBUF)]
    zidx_v, cnts_v = next(it), next(it)
    gsems = [next(it) for _ in range(_NBUF)]
    ssems = [next(it) for _ in range(_NBUF)]
    msem = next(it)
    acc = next(it)
    rows0 = rows[0]

    cid = lax.axis_index("c")
    sid = lax.axis_index("s")
    base = cid * _HALF
    zidx_v[...] = jnp.zeros((16,), jnp.int32)

    z = jnp.zeros((16,), jnp.float32)

    def zrow(i, carry):
        rows0[i, pl.ds(0, 16)] = z
        rows0[i, pl.ds(16, 16)] = z
        return carry

    lax.fori_loop(0, _K, zrow, 0)

    def zacc(j, carry):
        pltpu.sync_copy(rows0, acc.at[pl.ds(sid * _ZSPAN + j * _K, _K)])
        return carry

    lax.fori_loop(0, _ZSPAN // _K, zacc, 0)
    zrem = _ZSPAN - (_ZSPAN // _K) * _K
    if zrem:
        pltpu.sync_copy(rows0.at[pl.ds(0, zrem)],
                        acc.at[pl.ds(sid * _ZSPAN + (_ZSPAN // _K) * _K, zrem)])
    plsc.subcore_barrier()

    def region(rr, carry):
        w = sid * 2 + rr
        row = pl.multiple_of((cid * _NW + w) * 16, 16)
        pltpu.sync_copy(counts.at[pl.ds(row, 16)], cnts_v)
        nblk = cnts_v[...][0]
        rbase = cid * _COREOFF + w * _RCAP

        def block(i, c2):
            off = pl.multiple_of(rbase + i * _BLK, _BLK)
            md = [pltpu.async_copy(dstp.at[pl.ds(off, _BLK)], dstb, msem),
                  pltpu.async_copy(valp.at[pl.ds(off, _BLK)], valb, msem),
                  pltpu.async_copy(srcp.at[pl.ds(off, _BLK)], srcb, msem)]
            for d in md:
                d.wait()
            gd = [pltpu.async_copy(emb.at[dstb.at[pl.ds(j * _K, _K)]],
                                   rows[j], gsems[j])
                  for j in range(_NBUF)]
            sd = []
            for j in range(_NBUF):
                gd[j].wait()
                for g in range(_K // 16):
                    sls[j][pl.ds(g * 16, 16)] = srcb[pl.ds(j * _K + g * 16,
                                                           16)]
                _scale_rows(rows[j], valb, zidx_v, _K, vbase=j * _K)
                sd.append(pltpu.async_copy(rows[j], acc.at[sls[j]],
                                           ssems[j], add=True))
            for d in sd:
                d.wait()
            return c2

        lax.fori_loop(0, nblk, block, 0)
        return carry

    lax.fori_loop(0, 2, region, 0)

    plsc.subcore_barrier()

    @pl.when(sid < _NS - 1)
    def _copy_main():
        pltpu.sync_copy(acc.at[pl.ds(sid * _CSPAN, _CSPAN)],
                        out.at[pl.ds(base + sid * _CSPAN, _CSPAN)])

    @pl.when(sid == _NS - 1)
    def _copy_last():
        pltpu.sync_copy(acc.at[pl.ds(15 * _CSPAN, _CLAST)],
                        out.at[pl.ds(base + 15 * _CSPAN, _CLAST)])


def _propagate(emb, srcp, dstp, valp, counts):
    mesh = plsc.VectorSubcoreMesh(core_axis_name="c", subcore_axis_name="s",
                                  num_cores=_NC, num_subcores=_NS)
    f = pl.kernel(
        _prop_body,
        out_type=jax.ShapeDtypeStruct((_N_NODES, _EMB), jnp.float32),
        mesh=mesh,
        scratch_types=[
            pltpu.VMEM((_BLK,), jnp.int32),
            pltpu.VMEM((_BLK,), jnp.float32),
            pltpu.VMEM((_BLK,), jnp.int32),
        ] + [pltpu.VMEM((_K, _EMB), jnp.float32)] * _NBUF
          + [pltpu.VMEM((_K,), jnp.int32)] * _NBUF
          + [
            pltpu.VMEM((16,), jnp.int32),
            pltpu.VMEM((16,), jnp.int32),
        ] + [pltpu.SemaphoreType.DMA] * (2 * _NBUF + 1)
          + [pltpu.VMEM_SHARED((_PAD, _EMB), jnp.float32)],
        compiler_params=pltpu.CompilerParams(use_tc_tiling_on_sc=False,
                                             needs_layout_passes=False),
    )
    return f(emb, srcp, dstp, valp, counts)


def _mean_body(a_ref, b_ref, c_ref, d_ref, o_ref):
    o_ref[...] = (a_ref[...] + b_ref[...] + c_ref[...] + d_ref[...]) * 0.25


def _mean4(a, b, c, d):
    blk = (2000, _EMB)
    spec = pl.BlockSpec(blk, lambda i: (i, 0))
    return pl.pallas_call(
        _mean_body,
        grid=(_N_NODES // blk[0],),
        in_specs=[spec] * 4,
        out_specs=spec,
        out_shape=jax.ShapeDtypeStruct((_N_NODES, _EMB), jnp.float32),
    )(a, b, c, d)


def kernel(user_emb, item_emb, edge_src, edge_dst, edge_val):
    e0 = jnp.concatenate([user_emb, item_emb], axis=0)
    srcp, dstp, valp, counts = _partition(edge_src, edge_dst, edge_val)
    e1 = _propagate(e0, srcp, dstp, valp, counts)
    e2 = _propagate(e1, srcp, dstp, valp, counts)
    e3 = _propagate(e2, srcp, dstp, valp, counts)
    m = _mean4(e0, e1, e2, e3)
    return m[:_HALF], m[_HALF:]
